# baseline (speedup 1.0000x reference)
import functools

import jax
import jax.numpy as jnp
from jax import lax
from jax.experimental import pallas as pl
from jax.experimental.pallas import tpu as pltpu
from jax.experimental.pallas import tpu_sc as plsc

C = 16
D = 32
V = 1000000
N = 16384
I = 4096

_CH = 128


def _gather_rows(table, idx3, n_workers, rows_per_w):
    nch = rows_per_w // _CH
    mesh = plsc.VectorSubcoreMesh(core_axis_name="c", subcore_axis_name="s")

    @functools.partial(
        pl.kernel,
        mesh=mesh,
        compiler_params=pltpu.CompilerParams(use_tc_tiling_on_sc=False),
        out_type=jax.ShapeDtypeStruct((N, D), jnp.float32),
        scratch_types=[
            pltpu.VMEM((nch, _CH), jnp.int32),
            pltpu.VMEM((rows_per_w, D), jnp.float32),
            pltpu.SemaphoreType.DMA,
        ],
    )
    def k(table_hbm, idx_hbm, out_hbm, idx_v, rows_v, sem):
        wid = lax.axis_index("s") * 2 + lax.axis_index("c")
        pltpu.sync_copy(idx_hbm.at[wid], idx_v)
        descs = [
            pltpu.async_copy(
                table_hbm.at[idx_v.at[j]],
                rows_v.at[pl.ds(j * _CH, _CH)],
                sem,
            )
            for j in range(nch)
        ]
        for d_ in descs:
            d_.wait()
        pltpu.sync_copy(rows_v, out_hbm.at[pl.ds(wid * rows_per_w, rows_per_w)])

    return k(table, idx3)


def _dense_ll(mu, r, anno2, conf2):
    def body(mu_ref, r_ref, anno_ref, conf_ref, out_ref):
        E = jnp.exp(mu_ref[...])
        expE = jnp.exp(E)
        R = r_ref[...]
        S = lax.dot_general(jnp.exp(R), expE,
                            (((1,), (1,)), ((), ())),
                            preferred_element_type=jnp.float32)
        onehot = (anno_ref[...] ==
                  lax.broadcasted_iota(jnp.int32, (N, D), 1)
                  ).astype(jnp.float32)
        a = jnp.sum(R * onehot, axis=1, keepdims=True)
        G = lax.dot_general(onehot, E,
                            (((1,), (1,)), ((), ())),
                            preferred_element_type=jnp.float32)
        out_ref[...] = conf_ref[...] * (G + a - jnp.log(S))

    return pl.pallas_call(
        body,
        out_shape=jax.ShapeDtypeStruct((N, C), jnp.float32),
    )(mu, r, anno2, conf2)


def _segment_sum(ll, items3, n_workers, rows_per_w):
    nch = rows_per_w // _CH
    out_rows_per_w = I // n_workers
    mesh = plsc.VectorSubcoreMesh(core_axis_name="c", subcore_axis_name="s",
                                  num_cores=1)

    @functools.partial(
        pl.kernel,
        mesh=mesh,
        compiler_params=pltpu.CompilerParams(use_tc_tiling_on_sc=False),
        out_type=jax.ShapeDtypeStruct((I, C), jnp.float32),
        scratch_types=[
            pltpu.VMEM((nch, _CH), jnp.int32),
            pltpu.VMEM((rows_per_w, C), jnp.float32),
            pltpu.VMEM((out_rows_per_w, C), jnp.float32),
            pltpu.VMEM_SHARED((I, C), jnp.float32),
            pltpu.SemaphoreType.DMA,
        ],
    )
    def k(ll_hbm, items_hbm, out_hbm, idx_v, vals_v, z_v, acc_sh, sem):
        wid = lax.axis_index("s")

        def zero_row(i, carry):
            z_v[i, :] = jnp.zeros((C,), jnp.float32)
            return carry

        lax.fori_loop(0, out_rows_per_w, zero_row, 0)
        pltpu.sync_copy(z_v, acc_sh.at[pl.ds(wid * out_rows_per_w,
                                             out_rows_per_w)])
        pltpu.sync_copy(items_hbm.at[wid], idx_v)
        pltpu.sync_copy(ll_hbm.at[pl.ds(wid * rows_per_w, rows_per_w)], vals_v)
        plsc.subcore_barrier()
        for j in range(nch):
            pltpu.sync_copy(vals_v.at[pl.ds(j * _CH, _CH)],
                            acc_sh.at[idx_v.at[j]], add=True)
        plsc.subcore_barrier()
        pltpu.sync_copy(acc_sh.at[pl.ds(wid * out_rows_per_w, out_rows_per_w)],
                        out_hbm.at[pl.ds(wid * out_rows_per_w,
                                         out_rows_per_w)])

    return k(ll, items3)


def kernel(mu, random_effects, anno, items, annotators, confidences):
    gw = 32
    idx3 = annotators.astype(jnp.int32).reshape(gw, (N // gw) // _CH, _CH)
    r = _gather_rows(random_effects, idx3, gw, N // gw)

    anno2 = anno.astype(jnp.int32).reshape(N, 1)
    conf2 = confidences.reshape(N, 1)
    ll = _dense_ll(mu, r, anno2, conf2)

    sw = 16
    items3 = items.astype(jnp.int32).reshape(sw, (N // sw) // _CH, _CH)
    out = _segment_sum(ll, items3, sw, N // sw)
    return out.T
